# baseline (speedup 1.0000x reference)
import functools

import jax
import jax.numpy as jnp
from jax.experimental import pallas as pl


def _mlp_mask_kernel(stacks_ref, mask_ref, w1_ref, b1_ref, w2_ref, b2_ref, out_ref):
    bi, n, s = stacks_ref.shape[1], stacks_ref.shape[2], stacks_ref.shape[3]
    x = stacks_ref[...].reshape(bi * n, s)
    h = jnp.dot(x, w1_ref[...], preferred_element_type=jnp.float32) + b1_ref[...]
    h = jnp.maximum(h, 0.0)
    o = jnp.dot(h, w2_ref[...], preferred_element_type=jnp.float32) + b2_ref[...]
    o4 = o.reshape(1, bi, n, o.shape[-1])
    out_ref[...] = o4 * mask_ref[...]


@functools.partial(jax.jit, static_argnames=("block_i",))
def _run(stacks, mask_i8, W1, b1, W2, b2, block_i=16):
    b, n, _, s = stacks.shape
    heads = W2.shape[1]
    hidden = W1.shape[1]
    grid = (b, n // block_i)
    return pl.pallas_call(
        _mlp_mask_kernel,
        grid=grid,
        in_specs=[
            pl.BlockSpec((1, block_i, n, s), lambda ib, ii: (ib, ii, 0, 0)),
            pl.BlockSpec((1, block_i, n, 1), lambda ib, ii: (ib, ii, 0, 0)),
            pl.BlockSpec((s, hidden), lambda ib, ii: (0, 0)),
            pl.BlockSpec((1, hidden), lambda ib, ii: (0, 0)),
            pl.BlockSpec((hidden, heads), lambda ib, ii: (0, 0)),
            pl.BlockSpec((1, heads), lambda ib, ii: (0, 0)),
        ],
        out_specs=pl.BlockSpec((1, block_i, n, heads), lambda ib, ii: (ib, ii, 0, 0)),
        out_shape=jax.ShapeDtypeStruct((b, n, n, heads), jnp.float32),
    )(stacks, mask_i8, W1, b1, W2, b2)


def kernel(stacks, mask, W1, b1, W2, b2):
    mask_f = mask.astype(jnp.float32)[..., None]
    b1r = b1.reshape(1, -1)
    b2r = b2.reshape(1, -1)
    return _run(stacks, mask_f, W1, b1r, W2, b2r)


# speedup vs baseline: 1.0817x; 1.0817x over previous
import functools

import jax
import jax.numpy as jnp
import numpy as np
from jax.experimental import pallas as pl

_PACK = 4


def _mlp_mask_kernel(x_ref, m_ref, e_ref, w1_ref, b1_ref, w2_ref, b2_ref, out_ref):
    x = x_ref[...]
    h = jnp.dot(x, w1_ref[...], preferred_element_type=jnp.float32) + b1_ref[...]
    h = jnp.maximum(h, 0.0)
    o = jnp.dot(h, w2_ref[...], preferred_element_type=jnp.float32) + b2_ref[...]
    m = jnp.dot(m_ref[...], e_ref[...], preferred_element_type=jnp.float32)
    out_ref[...] = o * m


@functools.partial(jax.jit, static_argnames=("block_r",))
def _run(xs, mf, W1b, b1b, W2b, b2b, expand, block_r=2048):
    rows, width = xs.shape
    hid = W1b.shape[1]
    grid = (rows // block_r,)
    return pl.pallas_call(
        _mlp_mask_kernel,
        grid=grid,
        in_specs=[
            pl.BlockSpec((block_r, width), lambda i: (i, 0)),
            pl.BlockSpec((block_r, _PACK), lambda i: (i, 0)),
            pl.BlockSpec((_PACK, width), lambda i: (0, 0)),
            pl.BlockSpec((width, hid), lambda i: (0, 0)),
            pl.BlockSpec((1, hid), lambda i: (0, 0)),
            pl.BlockSpec((hid, width), lambda i: (0, 0)),
            pl.BlockSpec((1, width), lambda i: (0, 0)),
        ],
        out_specs=pl.BlockSpec((block_r, width), lambda i: (i, 0)),
        out_shape=jax.ShapeDtypeStruct((rows, width), jnp.float32),
    )(xs, mf, expand, W1b, b1b, W2b, b2b)


def kernel(stacks, mask, W1, b1, W2, b2):
    b, n, _, s = stacks.shape
    hidden = W1.shape[1]
    heads = W2.shape[1]
    rows = b * n * n // _PACK
    width = _PACK * s

    xs = stacks.reshape(rows, width)
    mf = mask.astype(jnp.float32).reshape(rows, _PACK)
    eye = jnp.eye(_PACK, dtype=jnp.float32)
    W1b = jnp.kron(eye, W1)
    W2b = jnp.kron(eye, W2)
    b1b = jnp.tile(b1, _PACK).reshape(1, -1)
    b2b = jnp.tile(b2, _PACK).reshape(1, -1)
    expand = jnp.kron(eye, jnp.ones((1, heads), jnp.float32))

    out = _run(xs, mf, W1b, b1b, W2b, b2b, expand)
    return out.reshape(b, n, n, heads)


# speedup vs baseline: 3.1853x; 2.9449x over previous
import functools

import jax
import jax.numpy as jnp
from jax.experimental import pallas as pl


def _mlp_mask_kernel(x_ref, m_ref, w1t_ref, b1_ref, w2t_ref, b2_ref, out_ref):
    ib = x_ref.shape[1]
    w1t = w1t_ref[...]
    w2t = w2t_ref[...]
    b1 = b1_ref[...]
    b2 = b2_ref[...]
    x = x_ref[0]
    m = m_ref[...]
    for i in range(ib):
        xi = x[i]
        h = jnp.dot(w1t, xi, preferred_element_type=jnp.float32) + b1
        h = jnp.maximum(h, 0.0)
        o = jnp.dot(w2t, h, preferred_element_type=jnp.float32) + b2
        mi = m[:, i, :].astype(jnp.float32)
        out_ref[0, i] = o * mi


@functools.partial(jax.jit, static_argnames=("block_i",))
def _run(xT, mask, W1t, b1c, W2t, b2c, block_i=8):
    b, n, s, nj = xT.shape
    hid = W1t.shape[0]
    heads = W2t.shape[0]
    grid = (b, n // block_i)
    return pl.pallas_call(
        _mlp_mask_kernel,
        grid=grid,
        in_specs=[
            pl.BlockSpec((1, block_i, s, nj), lambda ib, ii: (ib, ii, 0, 0)),
            pl.BlockSpec((1, block_i, nj), lambda ib, ii: (ib, ii, 0)),
            pl.BlockSpec((hid, s), lambda ib, ii: (0, 0)),
            pl.BlockSpec((hid, 1), lambda ib, ii: (0, 0)),
            pl.BlockSpec((heads, hid), lambda ib, ii: (0, 0)),
            pl.BlockSpec((heads, 1), lambda ib, ii: (0, 0)),
        ],
        out_specs=pl.BlockSpec((1, block_i, s, nj), lambda ib, ii: (ib, ii, 0, 0)),
        out_shape=jax.ShapeDtypeStruct((b, n, s, nj), jnp.float32),
    )(xT, mask, W1t, b1c, W2t, b2c)


def kernel(stacks, mask, W1, b1, W2, b2):
    xT = jnp.swapaxes(stacks, 2, 3)
    W1t = W1.T
    W2t = W2.T
    b1c = b1.reshape(-1, 1)
    b2c = b2.reshape(-1, 1)
    outT = _run(xT, mask, W1t, b1c, W2t, b2c)
    return jnp.swapaxes(outT, 2, 3)
